# Initial kernel scaffold; baseline (speedup 1.0000x reference)
import jax
import jax.numpy as jnp
from jax.experimental import pallas as pl


def kernel(node_feats, edge_feats, global_ids, edge_index, W_node, b_node, g_node, beta_node, W_edge, b_edge, g_edge, beta_edge, tpe, W1, b1, g1, beta1, W2, b2):
    raise NotImplementedError("write your pallas kernel here")


# speedup vs baseline: 55.7684x; 55.7684x over previous
import jax
import jax.numpy as jnp
from jax.experimental import pallas as pl
from jax.experimental.pallas import tpu as pltpu
from jax.experimental.pallas import tpu_sc as plsc


_LN_EPS = 1e-5


def _dot(a, b):
    return jax.lax.dot_general(a, b, (((1,), (0,)), ((), ())),
                               preferred_element_type=jnp.float32)


def _ln_rows(y):
    mu = jnp.mean(y, axis=-1, keepdims=True)
    var = jnp.mean((y - mu) ** 2, axis=-1, keepdims=True)
    return (y - mu) / jnp.sqrt(var + _LN_EPS)


def _node_enc_body(x_ref, w_ref, b_ref, g_ref, beta_ref, tpe_ref, o_ref):
    x = x_ref[0]
    y = _dot(x, w_ref[...]) + b_ref[0]
    o_ref[0] = _ln_rows(y) * g_ref[0] + beta_ref[0] + tpe_ref[0, 0]


def _head_body(e_ref, gs_ref, gd_ref, we_ref, be_ref, ge_ref, betae_ref,
               w1_ref, b1_ref, g1_ref, beta1_ref, w2_ref, b2_ref, o_ref):
    h_dim = we_ref.shape[1]
    e = e_ref[0]
    se = _ln_rows(_dot(e, we_ref[...]) + be_ref[0]) * ge_ref[0] + betae_ref[0]
    gs = gs_ref[0, 0]
    gd = gd_ref[0, 0]
    z = (_dot(se, w1_ref[0:h_dim, :])
         + _dot(gs, w1_ref[h_dim:2 * h_dim, :])
         + _dot(gd, w1_ref[2 * h_dim:3 * h_dim, :])
         + b1_ref[0])
    z = _ln_rows(z) * g1_ref[0] + beta1_ref[0]
    h = z * 0.5 * (1.0 + jax.lax.erf(z * (2.0 ** -0.5)))
    o_ref[0] = _dot(h, w2_ref[...]) + b2_ref[0]


def _sc_gather(table, idx_flat, h_dim):
    num_idx = idx_flat.shape[1]
    window = 128
    mesh = plsc.VectorSubcoreMesh(core_axis_name="core",
                                  subcore_axis_name="subcore")

    @pl.kernel(out_type=jax.ShapeDtypeStruct((num_idx, h_dim), table.dtype),
               mesh=mesh)
    def gather_kernel(table_hbm, idx_hbm, out_hbm):
        def body(i_vmem, o_vmem):
            pltpu.sync_copy(table_hbm.at[i_vmem.at[0]], o_vmem)

        pltpu.emit_pipeline(
            body,
            grid=(num_idx // window,),
            in_specs=[pl.BlockSpec((1, window), lambda i: (0, i))],
            out_specs=[pl.BlockSpec((window, h_dim), lambda i: (i, 0))],
            core_axis_name=("core", "subcore"),
            dimension_semantics=(pltpu.PARALLEL,),
        )(idx_hbm, out_hbm)

    return gather_kernel(table, idx_flat)


def kernel(node_feats, edge_feats, global_ids, edge_index, W_node, b_node,
           g_node, beta_node, W_edge, b_edge, g_edge, beta_edge, tpe, W1, b1,
           g1, beta1, W2, b2):
    T, N, NODE_IN = node_feats.shape
    _, E, EDGE_IN = edge_feats.shape
    H = W_node.shape[1]
    C = W2.shape[1]

    b_node2 = b_node.reshape(1, H)
    g_node2 = g_node.reshape(1, H)
    beta_node2 = beta_node.reshape(1, H)
    b_edge2 = b_edge.reshape(1, H)
    g_edge2 = g_edge.reshape(1, H)
    beta_edge2 = beta_edge.reshape(1, H)
    b1_2 = b1.reshape(1, 2 * H)
    g1_2 = g1.reshape(1, 2 * H)
    beta1_2 = beta1.reshape(1, 2 * H)
    b2_2 = b2.reshape(1, C)

    BN = 2000
    node_out = pl.pallas_call(
        _node_enc_body,
        grid=(T, N // BN),
        in_specs=[
            pl.BlockSpec((1, BN, NODE_IN), lambda t, i: (t, i, 0)),
            pl.BlockSpec((NODE_IN, H), lambda t, i: (0, 0)),
            pl.BlockSpec((1, H), lambda t, i: (0, 0)),
            pl.BlockSpec((1, H), lambda t, i: (0, 0)),
            pl.BlockSpec((1, H), lambda t, i: (0, 0)),
            pl.BlockSpec((1, 1, H), lambda t, i: (t, 0, 0)),
        ],
        out_specs=pl.BlockSpec((1, BN, H), lambda t, i: (t, i, 0)),
        out_shape=jax.ShapeDtypeStruct((T, N, H), jnp.float32),
    )(node_feats, W_node, b_node2, g_node2, beta_node2, tpe.reshape(T, 1, H))

    idx32 = edge_index.astype(jnp.int32)
    offs = (jnp.arange(T, dtype=jnp.int32) * N).reshape(T, 1, 1)
    idx_flat = (idx32 + offs).reshape(1, T * 2 * E)
    gathered = _sc_gather(node_out.reshape(T * N, H), idx_flat, H)
    gathered = gathered.reshape(T, 2, E, H)

    BE = 2048
    out = pl.pallas_call(
        _head_body,
        grid=(T, E // BE),
        in_specs=[
            pl.BlockSpec((1, BE, EDGE_IN), lambda t, i: (t, i, 0)),
            pl.BlockSpec((1, 1, BE, H), lambda t, i: (t, 0, i, 0)),
            pl.BlockSpec((1, 1, BE, H), lambda t, i: (t, 1, i, 0)),
            pl.BlockSpec((EDGE_IN, H), lambda t, i: (0, 0)),
            pl.BlockSpec((1, H), lambda t, i: (0, 0)),
            pl.BlockSpec((1, H), lambda t, i: (0, 0)),
            pl.BlockSpec((1, H), lambda t, i: (0, 0)),
            pl.BlockSpec((3 * H, 2 * H), lambda t, i: (0, 0)),
            pl.BlockSpec((1, 2 * H), lambda t, i: (0, 0)),
            pl.BlockSpec((1, 2 * H), lambda t, i: (0, 0)),
            pl.BlockSpec((1, 2 * H), lambda t, i: (0, 0)),
            pl.BlockSpec((2 * H, C), lambda t, i: (0, 0)),
            pl.BlockSpec((1, C), lambda t, i: (0, 0)),
        ],
        out_specs=pl.BlockSpec((1, BE, C), lambda t, i: (t, i, 0)),
        out_shape=jax.ShapeDtypeStruct((T, E, C), jnp.float32),
    )(edge_feats, gathered, gathered, W_edge, b_edge2, g_edge2, beta_edge2,
      W1, b1_2, g1_2, beta1_2, W2, b2_2)

    return out
